# Initial kernel scaffold; baseline (speedup 1.0000x reference)
import jax
import jax.numpy as jnp
from jax.experimental import pallas as pl


def kernel(x, pos, fps_pos, batch, frac_pos, trans_vec, scale, W1, b1, W2, b2):
    raise NotImplementedError("write your pallas kernel here")


# speedup vs baseline: 9.4323x; 9.4323x over previous
import jax
import jax.numpy as jnp
from jax import lax
from jax.experimental import pallas as pl
from jax.experimental.pallas import tpu as pltpu

B = 16
NP = 512
DF = 128
DH = 128
R = 0.25
KCAP = 64
TQ = 64
BS_ITERS = 26

_NT = (((1,), (1,)), ((), ()))
_NN = (((1,), (0,)), ((), ()))


def _dot(a, b, dn=_NN, prec=None):
    return lax.dot_general(a, b, dimension_numbers=dn,
                           preferred_element_type=jnp.float32,
                           precision=prec)


def _split(v):
    hi = v.astype(jnp.bfloat16)
    lo = (v - hi.astype(jnp.float32)).astype(jnp.bfloat16)
    return hi, lo


def _dot_x(a, b):
    ah, al = _split(a)
    bh, bl = _split(b)
    return _dot(ah, bh) + _dot(ah, bl) + _dot(al, bh)


def _select_kernel(fp_ref, fpt_ref, t_ref, scale_ref, slot_ref):
    f32 = jnp.float32

    fp = fp_ref[0]
    fpt = fpt_ref[0]
    tm = t_ref[0]

    diffs = [(fpt[m:m + 1, :] - fp[:, m:m + 1])
             .astype(jnp.bfloat16).astype(f32) for m in range(3)]
    tmb = tm.astype(jnp.bfloat16).astype(f32)
    d = None
    for k in range(3):
        tk = (diffs[0] * tmb[0, k] + diffs[1] * tmb[1, k]) + diffs[2] * tmb[2, k]
        sq = tk * tk
        d = sq if d is None else d + sq

    scale = scale_ref[0, 0, 0]
    rad2 = (R / scale) * (R / scale)
    inrad = d < rad2
    kf = jnp.float32(KCAP)

    total = jnp.sum(jnp.where(inrad, 1.0, 0.0), axis=1, keepdims=True)

    def bs_body(_, lohi):
        lo, hi = lohi
        mid = 0.5 * (lo + hi)
        cnt = jnp.sum(jnp.where(inrad & (d <= mid), 1.0, 0.0),
                      axis=1, keepdims=True)
        ok = cnt <= kf
        return jnp.where(ok, mid, lo), jnp.where(ok, hi, mid)

    lo, hi = lax.fori_loop(
        0, BS_ITERS, bs_body,
        (jnp.zeros((NP, 1), f32), jnp.full((NP, 1), rad2, f32)))

    use_bs = total > kf
    s_lo = inrad & (d <= lo)
    m = jnp.sum(jnp.where(s_lo, 1.0, 0.0), axis=1, keepdims=True)

    ut = (lax.broadcasted_iota(jnp.int32, (NP, NP), 0)
          <= lax.broadcasted_iota(jnp.int32, (NP, NP), 1)).astype(f32)

    bnd = inrad & (d <= hi) & (~s_lo)
    rank_b = _dot(jnp.where(bnd, 1.0, 0.0), ut)
    s_b = bnd & (rank_b <= (kf - m))
    sel = (use_bs & (s_lo | s_b)) | ((~use_bs) & inrad)

    s_f = jnp.where(sel, 1.0, 0.0)
    rank = _dot(s_f, ut)
    slot_ref[0] = jnp.where(sel, rank.astype(jnp.int32) - 1,
                            jnp.int32(1 << 30))


def _conv_kernel(slot_ref, x_ref, pos_ref,
                 w1x_ref, w1p_ref, b1_ref, w2_ref, b2_ref, out_ref):
    f32 = jnp.float32

    w = _dot(pos_ref[0], w1p_ref[...])
    a = _dot(x_ref[0], w1x_ref[...]) + w
    a_hi, a_lo = _split(a)

    b1 = b1_ref[...]
    b2 = b2_ref[...]
    w2 = w2_ref[...]

    for t in range(NP // TQ):
        sl = slice(t * TQ, (t + 1) * TQ)
        slot_t = slot_ref[0][sl]
        w_t = w[sl]

        slots = lax.broadcasted_iota(jnp.int32, (TQ, KCAP, NP), 1)
        g = (slot_t[:, None, :] == slots).astype(jnp.bfloat16)
        gm = g.reshape(TQ * KCAP, NP)
        ga = _dot(gm, a_hi) + _dot(gm, a_lo)

        pre = ga.reshape(TQ, KCAP, DH) - w_t[:, None, :] + b1[None]
        h1 = jnp.maximum(pre, 0.0)
        h = _dot(h1.reshape(TQ * KCAP, DH), w2)
        filled = jnp.sum(gm.astype(f32), axis=1, keepdims=True)
        h = h + (filled - 1.0) * 2e30
        hm = h.reshape(TQ, KCAP, DH)
        o = jnp.max(hm, axis=1) + b2
        o = jnp.where(o > -1e29, o, 0.0)
        out_ref[0, sl, :] = o


def _select_call(fp_pad, fpt_pad, t_pad, scale3):
    return pl.pallas_call(
        _select_kernel,
        grid=(B,),
        in_specs=[
            pl.BlockSpec((1, NP, 128), lambda b: (b, 0, 0)),
            pl.BlockSpec((1, 8, NP), lambda b: (b, 0, 0)),
            pl.BlockSpec((1, 128, 128), lambda b: (b, 0, 0)),
            pl.BlockSpec((1, 1, 1), lambda b: (b, 0, 0)),
        ],
        out_specs=pl.BlockSpec((1, NP, NP), lambda b: (b, 0, 0)),
        out_shape=jax.ShapeDtypeStruct((B, NP, NP), jnp.int32),
        compiler_params=pltpu.CompilerParams(
            dimension_semantics=("arbitrary",)),
    )(fp_pad, fpt_pad, t_pad, scale3)


def _conv_call(slot, x3, pos_pad, w1x, w1p, b1r, w2m, b2r):
    const2 = lambda b: (0, 0)
    return pl.pallas_call(
        _conv_kernel,
        grid=(B,),
        in_specs=[
            pl.BlockSpec((1, NP, NP), lambda b: (b, 0, 0)),
            pl.BlockSpec((1, NP, DF), lambda b: (b, 0, 0)),
            pl.BlockSpec((1, NP, 128), lambda b: (b, 0, 0)),
            pl.BlockSpec((DF, DH), const2),
            pl.BlockSpec((128, DH), const2),
            pl.BlockSpec((1, DH), const2),
            pl.BlockSpec((DH, DH), const2),
            pl.BlockSpec((1, DH), const2),
        ],
        out_specs=pl.BlockSpec((1, NP, DH), lambda b: (b, 0, 0)),
        out_shape=jax.ShapeDtypeStruct((B, NP, DH), jnp.float32),
        compiler_params=pltpu.CompilerParams(
            dimension_semantics=("arbitrary",)),
    )(slot, x3, pos_pad, w1x, w1p, b1r, w2m, b2r)


def kernel(x, pos, fps_pos, batch, frac_pos, trans_vec, scale,
           W1, b1, W2, b2):
    del fps_pos, batch
    f32 = jnp.float32
    n = B * NP

    fp_pad = jnp.zeros((n, 128), f32).at[:, :3].set(frac_pos).reshape(B, NP, 128)
    fpt_pad = jnp.zeros((B, 8, NP), f32).at[:, :3, :].set(
        frac_pos.reshape(B, NP, 3).transpose(0, 2, 1))
    pos_pad = jnp.zeros((n, 128), f32).at[:, :3].set(pos).reshape(B, NP, 128)
    t_pad = jnp.zeros((B, 128, 128), f32).at[:, :3, :3].set(trans_vec)
    x3 = x.reshape(B, NP, DF)
    scale3 = scale.reshape(B, 1, 1)

    w1x = W1[:DF]
    w1p = jnp.zeros((128, DH), f32).at[:3].set(W1[DF:DF + 3])
    b1r = b1.reshape(1, DH)
    b2r = b2.reshape(1, DH)

    slot = _select_call(fp_pad, fpt_pad, t_pad, scale3)
    out = _conv_call(slot, x3, pos_pad, w1x, w1p, b1r, W2, b2r)
    return out.reshape(n, DH)
